# Initial kernel scaffold; baseline (speedup 1.0000x reference)
import jax
import jax.numpy as jnp
from jax.experimental import pallas as pl


def kernel(hidden_states, routing_weights, gate_up_proj, gate_up_proj_bias, down_proj, down_proj_bias, lora_gate_up_A, lora_gate_up_B, lora_down_A, lora_down_B, router_indices):
    raise NotImplementedError("write your pallas kernel here")


# speedup vs baseline: 3.5632x; 3.5632x over previous
import functools

import jax
import jax.numpy as jnp
from jax.experimental import pallas as pl
from jax.experimental.pallas import tpu as pltpu

LIMIT = 7.0
ACT_ALPHA = 1.702
TB = 128


def _dense_body(x_ref, wg_ref, wu_ref, w2_ref, bg_ref, bu_ref, b2_ref,
                idx_ref, rw_ref, out_ref):
    e = pl.program_id(0)
    i = pl.program_id(1)
    x = x_ref[...]
    gate = jnp.dot(x, wg_ref[0], preferred_element_type=jnp.float32) + bg_ref[0]
    up = jnp.dot(x, wu_ref[0], preferred_element_type=jnp.float32) + bu_ref[0]
    gate = jnp.minimum(gate, LIMIT)
    up = jnp.clip(up, -LIMIT, LIMIT)
    glu = gate * (1.0 / (1.0 + jnp.exp(-ACT_ALPHA * gate)))
    gated = ((up + 1.0) * glu).astype(jnp.bfloat16)
    o = jnp.dot(gated, w2_ref[0], preferred_element_type=jnp.float32) + b2_ref[0]
    idx = idx_ref[0]
    rw = rw_ref[0]
    w = jnp.sum(jnp.where(idx == e, rw, 0.0), axis=1)
    contrib = o * w[:, None]
    rows = pl.ds(i * TB, TB)

    @pl.when(e == 0)
    def _init():
        out_ref[rows, :] = contrib

    @pl.when(e > 0)
    def _acc():
        out_ref[rows, :] += contrib


def kernel(hidden_states, routing_weights, gate_up_proj, gate_up_proj_bias,
           down_proj, down_proj_bias, lora_gate_up_A, lora_gate_up_B,
           lora_down_A, lora_down_B, router_indices):
    B_SZ, S, H = hidden_states.shape
    E, _, D = gate_up_proj.shape
    F = D // 2
    R = lora_gate_up_A.shape[1] // H
    scaling = 1.0 / R
    T = B_SZ * S
    NT = T // TB

    x = hidden_states.reshape(T, H)
    A1 = lora_gate_up_A.reshape(E, H, R)
    B1 = lora_gate_up_B.reshape(E, R, D)
    w1_eff = gate_up_proj + jnp.einsum('ehr,erd->ehd', A1, B1,
                                       preferred_element_type=jnp.float32) * scaling
    A2 = lora_down_A.reshape(E, F, R)
    B2 = lora_down_B.reshape(E, R, H)
    w2_eff = down_proj + jnp.einsum('efr,erh->efh', A2, B2,
                                    preferred_element_type=jnp.float32) * scaling
    wg = w1_eff[:, :, 0::2].astype(jnp.bfloat16)
    wu = w1_eff[:, :, 1::2].astype(jnp.bfloat16)
    w2 = w2_eff.astype(jnp.bfloat16)
    bg = gate_up_proj_bias[:, 0::2].reshape(E, 1, F)
    bu = gate_up_proj_bias[:, 1::2].reshape(E, 1, F)
    b2 = down_proj_bias.reshape(E, 1, H)
    xb = x.astype(jnp.bfloat16)
    idx3 = router_indices.reshape(NT, TB, -1)
    rw3 = routing_weights.reshape(NT, TB, -1)
    K = idx3.shape[-1]

    out = pl.pallas_call(
        _dense_body,
        grid=(E, NT),
        in_specs=[
            pl.BlockSpec((TB, H), lambda e, i: (i, 0)),
            pl.BlockSpec((1, H, F), lambda e, i: (e, 0, 0)),
            pl.BlockSpec((1, H, F), lambda e, i: (e, 0, 0)),
            pl.BlockSpec((1, F, H), lambda e, i: (e, 0, 0)),
            pl.BlockSpec((1, 1, F), lambda e, i: (e, 0, 0)),
            pl.BlockSpec((1, 1, F), lambda e, i: (e, 0, 0)),
            pl.BlockSpec((1, 1, H), lambda e, i: (e, 0, 0)),
            pl.BlockSpec((1, TB, K), lambda e, i: (i, 0, 0)),
            pl.BlockSpec((1, TB, K), lambda e, i: (i, 0, 0)),
        ],
        out_specs=pl.BlockSpec((T, H), lambda e, i: (0, 0)),
        out_shape=jax.ShapeDtypeStruct((T, H), jnp.float32),
    )(xb, wg, wu, w2, bg, bu, b2, idx3, rw3)
    return out.reshape(B_SZ, S, H)


# speedup vs baseline: 10.2136x; 2.8664x over previous
import jax
import jax.numpy as jnp
from jax.experimental import pallas as pl
from jax.experimental.pallas import tpu as pltpu

LIMIT = 7.0
ACT_ALPHA = 1.702
TB = 128


def _dense_body(x_ref, w1_ref, w2_ref, b1_ref, b2_ref, idx_ref, rw_ref,
                out_ref):
    e = pl.program_id(0)
    i = pl.program_id(1)
    x = x_ref[...]
    gu = jnp.dot(x, w1_ref[0], preferred_element_type=jnp.float32) + b1_ref[0]
    up = jnp.clip(pltpu.roll(gu, gu.shape[1] - 1, 1), -LIMIT, LIMIT)
    gate = jnp.minimum(gu, LIMIT)
    glu = gate * (1.0 / (1.0 + jnp.exp(-ACT_ALPHA * gate)))
    gated = ((up + 1.0) * glu).astype(jnp.bfloat16)
    o = jnp.dot(gated, w2_ref[0], preferred_element_type=jnp.float32) + b2_ref[0]
    idx = idx_ref[0]
    rw = rw_ref[0]
    w = jnp.sum(jnp.where(idx == e, rw, 0.0), axis=1)
    contrib = o * w[:, None]
    rows = pl.ds(i * TB, TB)

    @pl.when(e == 0)
    def _init():
        out_ref[rows, :] = contrib

    @pl.when(e > 0)
    def _acc():
        out_ref[rows, :] += contrib


def kernel(hidden_states, routing_weights, gate_up_proj, gate_up_proj_bias,
           down_proj, down_proj_bias, lora_gate_up_A, lora_gate_up_B,
           lora_down_A, lora_down_B, router_indices):
    B_SZ, S, H = hidden_states.shape
    E, _, D = gate_up_proj.shape
    F = D // 2
    R = lora_gate_up_A.shape[1] // H
    scaling = 1.0 / R
    T = B_SZ * S
    NT = T // TB

    x = hidden_states.reshape(T, H)
    A1 = lora_gate_up_A.reshape(E, H, R)
    B1 = lora_gate_up_B.reshape(E, R, D)
    w1_eff = (gate_up_proj + jnp.einsum('ehr,erd->ehd', A1, B1,
                                        preferred_element_type=jnp.float32)
              * scaling).astype(jnp.bfloat16)
    A2 = lora_down_A.reshape(E, F, R)
    B2 = lora_down_B.reshape(E, R, H)
    w2_eff = (down_proj + jnp.einsum('efr,erh->efh', A2, B2,
                                     preferred_element_type=jnp.float32)
              * scaling).astype(jnp.bfloat16)
    w2i = jnp.stack([w2_eff, jnp.zeros_like(w2_eff)], axis=2).reshape(E, D, H)
    b1 = gate_up_proj_bias.reshape(E, 1, D)
    b2 = down_proj_bias.reshape(E, 1, H)
    xb = x.astype(jnp.bfloat16)
    idx3 = router_indices.reshape(NT, TB, -1)
    rw3 = routing_weights.reshape(NT, TB, -1)
    K = idx3.shape[-1]

    out = pl.pallas_call(
        _dense_body,
        grid=(E, NT),
        in_specs=[
            pl.BlockSpec((TB, H), lambda e, i: (i, 0)),
            pl.BlockSpec((1, H, D), lambda e, i: (e, 0, 0)),
            pl.BlockSpec((1, D, H), lambda e, i: (e, 0, 0)),
            pl.BlockSpec((1, 1, D), lambda e, i: (e, 0, 0)),
            pl.BlockSpec((1, 1, H), lambda e, i: (e, 0, 0)),
            pl.BlockSpec((1, TB, K), lambda e, i: (i, 0, 0)),
            pl.BlockSpec((1, TB, K), lambda e, i: (i, 0, 0)),
        ],
        out_specs=pl.BlockSpec((T, H), lambda e, i: (0, 0)),
        out_shape=jax.ShapeDtypeStruct((T, H), jnp.float32),
    )(xb, w1_eff, w2i, b1, b2, idx3, rw3)
    return out.reshape(B_SZ, S, H)
